# baseline (speedup 1.0000x reference)
import functools

import jax
import jax.numpy as jnp
from jax import lax
from jax.experimental import pallas as pl
from jax.experimental.pallas import tpu as pltpu
from jax.experimental.pallas import tpu_sc as plsc

S, B, D = 2048, 2, 1024
E, K, DFF = 8, 2, 4096
T = S * B
C = (T * K) // E
EP = 128
L = 16
ZROW = E * C
_MESH = dict(core_axis_name="c", subcore_axis_name="s",
             num_cores=2, num_subcores=16)


def _gating_body(x_ref, wg_ref, probsT_ref, combT_ref, topiT_ref, laux_ref,
                 cnt_ref):
    x = x_ref[...]
    logits = jnp.dot(x, wg_ref[...], preferred_element_type=jnp.float32)
    lane = lax.broadcasted_iota(jnp.int32, (T, EP), 1)
    real = lane < E
    logits = jnp.where(real, logits, -1e30)
    m = jnp.max(logits, axis=1, keepdims=True)
    ex = jnp.exp(logits - m)
    probs = ex / jnp.sum(ex, axis=1, keepdims=True)

    p1 = jnp.max(probs, axis=1, keepdims=True)
    i1 = jnp.min(jnp.where(probs == p1, lane, EP), axis=1, keepdims=True)
    pm = jnp.where((lane == i1) | ~real, -1.0, probs)
    p2 = jnp.max(pm, axis=1, keepdims=True)
    i2 = jnp.min(jnp.where(pm == p2, lane, EP), axis=1, keepdims=True)
    combine = jnp.where(lane == i1, p1, 0.0) + jnp.where(lane == i2, p2, 0.0)

    one1 = (lane == i1)
    one2 = (lane == i2)
    me = jnp.sum(probs, axis=0, keepdims=True) * (1.0 / T)
    ce = jnp.sum(one1.astype(jnp.float32), axis=0, keepdims=True) * (1.0 / T)
    laux_ref[...] = jnp.reshape(E * jnp.sum(me * ce), (1, 1))
    cnt_ref[...] = jnp.sum(one1.astype(jnp.int32) + one2.astype(jnp.int32),
                           axis=0, keepdims=True)

    probsT_ref[...] = jnp.transpose(probs)[:E, :]
    combT_ref[...] = jnp.transpose(combine)[:E, :]
    ti = jnp.where(lane == 0, i1, i2)
    tif = lax.bitcast_convert_type(ti, jnp.float32)
    topiT_ref[...] = lax.bitcast_convert_type(
        jnp.transpose(tif)[:K, :], jnp.int32)


_gating = pl.pallas_call(
    _gating_body,
    out_shape=(
        jax.ShapeDtypeStruct((E, T), jnp.float32),
        jax.ShapeDtypeStruct((E, T), jnp.float32),
        jax.ShapeDtypeStruct((K, T), jnp.int32),
        jax.ShapeDtypeStruct((1, 1), jnp.float32),
        jax.ShapeDtypeStruct((1, EP), jnp.int32),
    ),
)


_PCH = 128
_NPCH = T // _PCH
_WCH = 256


def _capacity_body(probsT_ref, combT_ref, pf_ref, wv_ref, posb, selb):
    pT = probsT_ref[...]

    def bs(_, lohi):
        lo, hi = lohi
        mid = (lo + hi) // 2
        midf = lax.bitcast_convert_type(mid, jnp.float32)
        cnt = jnp.sum(jnp.where(pT > midf, 1.0, 0.0), axis=1, keepdims=True)
        ge = cnt >= float(C)
        return jnp.where(ge, mid + 1, lo), jnp.where(ge, hi, mid)

    lo0 = jnp.zeros((E, 1), jnp.int32)
    hi0 = jnp.full((E, 1), 0x3F800000, jnp.int32)
    lo, _ = lax.fori_loop(0, 31, bs, (lo0, hi0))
    theta = lax.bitcast_convert_type(lo, jnp.float32)
    cnt_gt = jnp.sum(jnp.where(pT > theta, 1.0, 0.0), axis=1, keepdims=True)
    needed = float(C) - cnt_gt

    r2 = lax.broadcasted_iota(jnp.int32, (_PCH, _PCH), 0)
    c2 = lax.broadcasted_iota(jnp.int32, (_PCH, _PCH), 1)
    ut = (r2 < c2).astype(jnp.float32)
    erow = lax.broadcasted_iota(jnp.int32, (E, _PCH), 0) * C

    def chunk(c, carry):
        eqc, selc = carry
        pg = probsT_ref[:, pl.ds(c * _PCH, _PCH)]
        m_gt = pg > theta
        m_eq = pg == theta
        eq_f = m_eq.astype(jnp.float32)
        rank = jnp.dot(eq_f, ut, preferred_element_type=jnp.float32) + eqc
        sel = m_gt | (m_eq & (rank < needed))
        sel_f = sel.astype(jnp.float32)
        pos = jnp.dot(sel_f, ut, preferred_element_type=jnp.float32) + selc
        pf = jnp.where(sel, erow + pos.astype(jnp.int32), ZROW)
        pf_ref[:, pl.ds(c * _PCH, _PCH)] = pf
        posb[:, pl.ds(c * _PCH, _PCH)] = pos
        selb[:, pl.ds(c * _PCH, _PCH)] = sel_f
        return (eqc + jnp.sum(eq_f, axis=1, keepdims=True),
                selc + jnp.sum(sel_f, axis=1, keepdims=True))

    z = jnp.zeros((E, 1), jnp.float32)
    lax.fori_loop(0, _NPCH, chunk, (z, z))

    for e in range(E):
        pos_col = jnp.transpose(posb[pl.ds(e, 1), :])
        sel_col = jnp.transpose(selb[pl.ds(e, 1), :])
        cv = combT_ref[pl.ds(e, 1), :]

        def wchunk(cc, _):
            cio = (lax.broadcasted_iota(jnp.int32, (1, _WCH), 1)
                   + cc * _WCH).astype(jnp.float32)
            mc = ((pos_col == cio) & (sel_col > 0.0)).astype(jnp.float32)
            wv_ref[pl.ds(e, 1), pl.ds(cc * _WCH, _WCH)] = jnp.dot(
                cv, mc, preferred_element_type=jnp.float32)
            return 0

        lax.fori_loop(0, C // _WCH, wchunk, 0)


_capacity = pl.pallas_call(
    _capacity_body,
    out_shape=(
        jax.ShapeDtypeStruct((E, T), jnp.int32),
        jax.ShapeDtypeStruct((E, C), jnp.float32),
    ),
    scratch_shapes=[
        pltpu.VMEM((E, T), jnp.float32),
        pltpu.VMEM((E, T), jnp.float32),
    ],
)


_TPW = T // 32
_RCH = 32


def _sel8(pfw, t_ref, i):
    ev = t_ref[pl.ds(i * L, L)]
    acc = jnp.full((L,), ZROW, jnp.int32)
    for e in range(E):
        acc = jnp.where(ev == e, pfw[e, pl.ds(i * L, L)], acc)
    return acc


def _dispatch_body(x_hbm, pf_hbm, topiT_hbm, xe_hbm, pfw, t0, t1, fidx,
                   rows0, rows1, lsem0, lsem1, ssem0, ssem1):
    wid = lax.axis_index("c") * 16 + lax.axis_index("s")
    base = wid * _TPW
    for e in range(E):
        pltpu.sync_copy(pf_hbm.at[e, pl.ds(base, _TPW)], pfw.at[e])
    pltpu.sync_copy(topiT_hbm.at[0, pl.ds(base, _TPW)], t0)
    pltpu.sync_copy(topiT_hbm.at[1, pl.ds(base, _TPW)], t1)

    def fstep(i, _):
        c = i // 2
        h = (i % 2) * L
        fidx[c, pl.ds(h, L)] = _sel8(pfw, t0, i)
        fidx[4 + c, pl.ds(h, L)] = _sel8(pfw, t1, i)
        return 0

    lax.fori_loop(0, _TPW // L, fstep, 0)

    nch = _TPW // _RCH
    bufs = (rows0, rows1)
    lsem = (lsem0, lsem1)
    ssem = (ssem0, ssem1)
    ld = [pltpu.async_copy(x_hbm.at[pl.ds(base, _RCH)], rows0, lsem0), None]
    sc = [None, None]
    for c in range(nch):
        b = c % 2
        ld[b].wait()
        if c + 1 < nch:
            if sc[1 - b] is not None:
                sc[1 - b][0].wait()
                sc[1 - b][1].wait()
                sc[1 - b] = None
            ld[1 - b] = pltpu.async_copy(
                x_hbm.at[pl.ds(base + (c + 1) * _RCH, _RCH)],
                bufs[1 - b], lsem[1 - b])
        sc[b] = (
            pltpu.async_copy(bufs[b], xe_hbm.at[fidx.at[c]], ssem[b]),
            pltpu.async_copy(bufs[b], xe_hbm.at[fidx.at[4 + c]], ssem[b]),
        )
    for ws in sc:
        if ws is not None:
            ws[0].wait()
            ws[1].wait()


_dispatch = functools.partial(
    pl.kernel,
    out_type=jax.ShapeDtypeStruct(((E + 1) * C, D), jnp.float32),
    mesh=plsc.VectorSubcoreMesh(**_MESH),
    scratch_types=[
        pltpu.VMEM((E, _TPW), jnp.int32),
        pltpu.VMEM((_TPW,), jnp.int32),
        pltpu.VMEM((_TPW,), jnp.int32),
        pltpu.VMEM((8, _RCH), jnp.int32),
        pltpu.VMEM((_RCH, D), jnp.float32),
        pltpu.VMEM((_RCH, D), jnp.float32),
        pltpu.SemaphoreType.DMA,
        pltpu.SemaphoreType.DMA,
        pltpu.SemaphoreType.DMA,
        pltpu.SemaphoreType.DMA,
    ],
)(_dispatch_body)


_JCH = 1024
_NJ = DFF // _JCH


def _mlp_body(x_ref, w1_ref, b1_ref, w2_ref, b2_ref, wv_ref, o_ref):
    e = pl.program_id(0)
    j = pl.program_id(1)

    @pl.when(e == E)
    def _():
        o_ref[...] = jnp.zeros((C, D), jnp.float32)

    @pl.when(e < E)
    def _():
        h = jnp.dot(x_ref[...].astype(jnp.bfloat16),
                    w1_ref[0].astype(jnp.bfloat16),
                    preferred_element_type=jnp.float32)
        h = jax.nn.gelu(h + b1_ref[0])
        part = jnp.dot(h.astype(jnp.bfloat16),
                       w2_ref[0].astype(jnp.bfloat16),
                       preferred_element_type=jnp.float32)

        @pl.when(j == 0)
        def _():
            o_ref[...] = part + b2_ref[0]

        @pl.when(j > 0)
        def _():
            o_ref[...] += part

        @pl.when(j == _NJ - 1)
        def _():
            o_ref[...] *= wv_ref[0]


_mlp = pl.pallas_call(
    _mlp_body,
    grid=(E + 1, _NJ),
    in_specs=[
        pl.BlockSpec((C, D), lambda e, j: (jnp.minimum(e, E - 1), 0)),
        pl.BlockSpec((1, D, _JCH), lambda e, j: (jnp.minimum(e, E - 1), 0, j)),
        pl.BlockSpec((1, 1, _JCH), lambda e, j: (jnp.minimum(e, E - 1), 0, j)),
        pl.BlockSpec((1, _JCH, D), lambda e, j: (jnp.minimum(e, E - 1), j, 0)),
        pl.BlockSpec((1, 1, D), lambda e, j: (jnp.minimum(e, E - 1), 0, 0)),
        pl.BlockSpec((1, C, 1), lambda e, j: (jnp.minimum(e, E - 1), 0, 0)),
    ],
    out_specs=pl.BlockSpec((C, D), lambda e, j: (e, 0)),
    out_shape=jax.ShapeDtypeStruct(((E + 1) * C, D), jnp.float32),
    compiler_params=pltpu.CompilerParams(
        dimension_semantics=("arbitrary", "arbitrary")),
)


_CCH = 16


def _combine_body(eout_hbm, pf_hbm, topiT_hbm, out_hbm, pfw, t0, t1, fidx,
                  bufA0, bufB0, bufA1, bufB1, gsem0, gsem1, wsem0, wsem1):
    wid = lax.axis_index("c") * 16 + lax.axis_index("s")
    base = wid * _TPW
    for e in range(E):
        pltpu.sync_copy(pf_hbm.at[e, pl.ds(base, _TPW)], pfw.at[e])
    pltpu.sync_copy(topiT_hbm.at[0, pl.ds(base, _TPW)], t0)
    pltpu.sync_copy(topiT_hbm.at[1, pl.ds(base, _TPW)], t1)

    nch = _TPW // _CCH

    def fstep(i, _):
        fidx[i, :] = _sel8(pfw, t0, i)
        fidx[nch + i, :] = _sel8(pfw, t1, i)
        return 0

    lax.fori_loop(0, _TPW // L, fstep, 0)

    bufA = (bufA0, bufA1)
    bufB = (bufB0, bufB1)
    gsem = (gsem0, gsem1)
    wsem = (wsem0, wsem1)
    gd = [(pltpu.async_copy(eout_hbm.at[fidx.at[0]], bufA0, gsem0),
           pltpu.async_copy(eout_hbm.at[fidx.at[nch]], bufB0, gsem0)), None]
    wd = [None, None]
    for c in range(nch):
        b = c % 2
        nb = 1 - b
        gd[b][0].wait()
        gd[b][1].wait()
        if c + 1 < nch:
            if wd[nb] is not None:
                wd[nb].wait()
                wd[nb] = None
            gd[nb] = (
                pltpu.async_copy(eout_hbm.at[fidx.at[c + 1]], bufA[nb],
                                 gsem[nb]),
                pltpu.async_copy(eout_hbm.at[fidx.at[nch + c + 1]], bufB[nb],
                                 gsem[nb]),
            )

        def addr(r, _):
            def addv(v, __):
                bufA[b][r, pl.ds(v * L, L)] += bufB[b][r, pl.ds(v * L, L)]
                return 0
            lax.fori_loop(0, D // L, addv, 0, unroll=8)
            return 0

        lax.fori_loop(0, _CCH, addr, 0)
        wd[b] = pltpu.async_copy(bufA[b],
                                 out_hbm.at[pl.ds(base + c * _CCH, _CCH)],
                                 wsem[b])
    for w in wd:
        if w is not None:
            w.wait()


_combine = functools.partial(
    pl.kernel,
    out_type=jax.ShapeDtypeStruct((T, D), jnp.float32),
    mesh=plsc.VectorSubcoreMesh(**_MESH),
    scratch_types=[
        pltpu.VMEM((E, _TPW), jnp.int32),
        pltpu.VMEM((_TPW,), jnp.int32),
        pltpu.VMEM((_TPW,), jnp.int32),
        pltpu.VMEM((2 * (_TPW // _CCH), _CCH), jnp.int32),
        pltpu.VMEM((_CCH, D), jnp.float32),
        pltpu.VMEM((_CCH, D), jnp.float32),
        pltpu.VMEM((_CCH, D), jnp.float32),
        pltpu.VMEM((_CCH, D), jnp.float32),
        pltpu.SemaphoreType.DMA,
        pltpu.SemaphoreType.DMA,
        pltpu.SemaphoreType.DMA,
        pltpu.SemaphoreType.DMA,
    ],
)(_combine_body)


def kernel(hidden_states, wg, w1, b1, w2, b2, now_training_process):
    x = hidden_states.reshape(T, D)
    wg_p = jnp.pad(wg, ((0, 0), (0, EP - E)))
    probsT, combT, topiT, laux, cnt = _gating(x, wg_p)
    pf, wvec = _capacity(probsT, combT)
    xe = _dispatch(x, pf, topiT)
    eout = _mlp(xe, w1, b1.reshape(E, 1, DFF), w2, b2.reshape(E, 1, D),
                wvec.reshape(E, C, 1))
    out = _combine(eout, pf, topiT)
    return (out.reshape(hidden_states.shape),
            laux.reshape(()),
            cnt[0, :E])


# speedup vs baseline: 1.0416x; 1.0416x over previous
import functools

import jax
import jax.numpy as jnp
from jax import lax
from jax.experimental import pallas as pl
from jax.experimental.pallas import tpu as pltpu
from jax.experimental.pallas import tpu_sc as plsc

S, B, D = 2048, 2, 1024
E, K, DFF = 8, 2, 4096
T = S * B
C = (T * K) // E
EP = 128
L = 16
ZROW = E * C
_MESH = dict(core_axis_name="c", subcore_axis_name="s",
             num_cores=2, num_subcores=16)


def _gating_body(x_ref, wg_ref, probsT_ref, combT_ref, topiT_ref, laux_ref,
                 cnt_ref):
    x = x_ref[...]
    logits = jnp.dot(x, wg_ref[...], preferred_element_type=jnp.float32)
    lane = lax.broadcasted_iota(jnp.int32, (T, EP), 1)
    real = lane < E
    logits = jnp.where(real, logits, -1e30)
    m = jnp.max(logits, axis=1, keepdims=True)
    ex = jnp.exp(logits - m)
    probs = ex / jnp.sum(ex, axis=1, keepdims=True)

    p1 = jnp.max(probs, axis=1, keepdims=True)
    i1 = jnp.min(jnp.where(probs == p1, lane, EP), axis=1, keepdims=True)
    pm = jnp.where((lane == i1) | ~real, -1.0, probs)
    p2 = jnp.max(pm, axis=1, keepdims=True)
    i2 = jnp.min(jnp.where(pm == p2, lane, EP), axis=1, keepdims=True)
    combine = jnp.where(lane == i1, p1, 0.0) + jnp.where(lane == i2, p2, 0.0)

    one1 = (lane == i1)
    one2 = (lane == i2)
    me = jnp.sum(probs, axis=0, keepdims=True) * (1.0 / T)
    ce = jnp.sum(one1.astype(jnp.float32), axis=0, keepdims=True) * (1.0 / T)
    laux_ref[...] = jnp.reshape(E * jnp.sum(me * ce), (1, 1))
    cnt_ref[...] = jnp.sum(one1.astype(jnp.int32) + one2.astype(jnp.int32),
                           axis=0, keepdims=True)

    probsT_ref[...] = jnp.transpose(probs)[:E, :]
    combT_ref[...] = jnp.transpose(combine)[:E, :]
    ti = jnp.where(lane == 0, i1, i2)
    tif = lax.bitcast_convert_type(ti, jnp.float32)
    topiT_ref[...] = lax.bitcast_convert_type(
        jnp.transpose(tif)[:K, :], jnp.int32)


_gating = pl.pallas_call(
    _gating_body,
    out_shape=(
        jax.ShapeDtypeStruct((E, T), jnp.float32),
        jax.ShapeDtypeStruct((E, T), jnp.float32),
        jax.ShapeDtypeStruct((K, T), jnp.int32),
        jax.ShapeDtypeStruct((1, 1), jnp.float32),
        jax.ShapeDtypeStruct((1, EP), jnp.int32),
    ),
)


_PCH = 128
_NPCH = T // _PCH
_WCH = 256


def _capacity_body(probsT_ref, combT_ref, pf_ref, wv_ref, posb, selb):
    pT = probsT_ref[...]

    def bs(_, lohi):
        lo, hi = lohi
        mid = (lo + hi) // 2
        midf = lax.bitcast_convert_type(mid, jnp.float32)
        cnt = jnp.sum(jnp.where(pT > midf, 1.0, 0.0), axis=1, keepdims=True)
        ge = cnt >= float(C)
        return jnp.where(ge, mid + 1, lo), jnp.where(ge, hi, mid)

    lo0 = jnp.zeros((E, 1), jnp.int32)
    hi0 = jnp.full((E, 1), 0x3F800000, jnp.int32)
    lo, _ = lax.fori_loop(0, 31, bs, (lo0, hi0))
    theta = lax.bitcast_convert_type(lo, jnp.float32)
    cnt_gt = jnp.sum(jnp.where(pT > theta, 1.0, 0.0), axis=1, keepdims=True)
    needed = float(C) - cnt_gt

    r2 = lax.broadcasted_iota(jnp.int32, (_PCH, _PCH), 0)
    c2 = lax.broadcasted_iota(jnp.int32, (_PCH, _PCH), 1)
    ut = (r2 < c2).astype(jnp.float32)
    erow = lax.broadcasted_iota(jnp.int32, (E, _PCH), 0) * C

    def chunk(c, carry):
        eqc, selc = carry
        pg = probsT_ref[:, pl.ds(c * _PCH, _PCH)]
        m_gt = pg > theta
        m_eq = pg == theta
        eq_f = m_eq.astype(jnp.float32)
        rank = jnp.dot(eq_f, ut, preferred_element_type=jnp.float32) + eqc
        sel = m_gt | (m_eq & (rank < needed))
        sel_f = sel.astype(jnp.float32)
        pos = jnp.dot(sel_f, ut, preferred_element_type=jnp.float32) + selc
        pf = jnp.where(sel, erow + pos.astype(jnp.int32), ZROW)
        pf_ref[:, pl.ds(c * _PCH, _PCH)] = pf
        posb[:, pl.ds(c * _PCH, _PCH)] = pos
        selb[:, pl.ds(c * _PCH, _PCH)] = sel_f
        return (eqc + jnp.sum(eq_f, axis=1, keepdims=True),
                selc + jnp.sum(sel_f, axis=1, keepdims=True))

    z = jnp.zeros((E, 1), jnp.float32)
    lax.fori_loop(0, _NPCH, chunk, (z, z))

    for e in range(E):
        pos_col = jnp.transpose(posb[pl.ds(e, 1), :])
        sel_col = jnp.transpose(selb[pl.ds(e, 1), :])
        cv = combT_ref[pl.ds(e, 1), :]

        def wchunk(cc, _):
            cio = (lax.broadcasted_iota(jnp.int32, (1, _WCH), 1)
                   + cc * _WCH).astype(jnp.float32)
            mc = ((pos_col == cio) & (sel_col > 0.0)).astype(jnp.float32)
            wv_ref[pl.ds(e, 1), pl.ds(cc * _WCH, _WCH)] = jnp.dot(
                cv, mc, preferred_element_type=jnp.float32)
            return 0

        lax.fori_loop(0, C // _WCH, wchunk, 0)


_capacity = pl.pallas_call(
    _capacity_body,
    out_shape=(
        jax.ShapeDtypeStruct((E, T), jnp.int32),
        jax.ShapeDtypeStruct((E, C), jnp.float32),
    ),
    scratch_shapes=[
        pltpu.VMEM((E, T), jnp.float32),
        pltpu.VMEM((E, T), jnp.float32),
    ],
)


_TPW = T // 32
_RCH = 32


def _sel8(pfw, t_ref, i):
    ev = t_ref[pl.ds(i * L, L)]
    acc = jnp.full((L,), ZROW, jnp.int32)
    for e in range(E):
        acc = jnp.where(ev == e, pfw[e, pl.ds(i * L, L)], acc)
    return acc


def _dispatch_body(x_hbm, pf_hbm, topiT_hbm, xe_hbm, pfw, t0, t1, fidx,
                   rows0, rows1, lsem0, lsem1, ssem0, ssem1):
    wid = lax.axis_index("c") * 16 + lax.axis_index("s")
    base = wid * _TPW
    pltpu.sync_copy(pf_hbm.at[:, pl.ds(base, _TPW)], pfw)
    pltpu.sync_copy(topiT_hbm.at[0, pl.ds(base, _TPW)], t0)
    pltpu.sync_copy(topiT_hbm.at[1, pl.ds(base, _TPW)], t1)

    def fstep(i, _):
        c = i // 2
        h = (i % 2) * L
        fidx[c, pl.ds(h, L)] = _sel8(pfw, t0, i)
        fidx[4 + c, pl.ds(h, L)] = _sel8(pfw, t1, i)
        return 0

    lax.fori_loop(0, _TPW // L, fstep, 0)

    nch = _TPW // _RCH
    bufs = (rows0, rows1)
    lsem = (lsem0, lsem1)
    ssem = (ssem0, ssem1)
    ld = [pltpu.async_copy(x_hbm.at[pl.ds(base, _RCH)], rows0, lsem0), None]
    sc = [None, None]
    for c in range(nch):
        b = c % 2
        ld[b].wait()
        if c + 1 < nch:
            if sc[1 - b] is not None:
                sc[1 - b][0].wait()
                sc[1 - b][1].wait()
                sc[1 - b] = None
            ld[1 - b] = pltpu.async_copy(
                x_hbm.at[pl.ds(base + (c + 1) * _RCH, _RCH)],
                bufs[1 - b], lsem[1 - b])
        sc[b] = (
            pltpu.async_copy(bufs[b], xe_hbm.at[fidx.at[c]], ssem[b]),
            pltpu.async_copy(bufs[b], xe_hbm.at[fidx.at[4 + c]], ssem[b]),
        )
    for ws in sc:
        if ws is not None:
            ws[0].wait()
            ws[1].wait()


_dispatch = functools.partial(
    pl.kernel,
    out_type=jax.ShapeDtypeStruct(((E + 1) * C, D), jnp.float32),
    mesh=plsc.VectorSubcoreMesh(**_MESH),
    scratch_types=[
        pltpu.VMEM((E, _TPW), jnp.int32),
        pltpu.VMEM((_TPW,), jnp.int32),
        pltpu.VMEM((_TPW,), jnp.int32),
        pltpu.VMEM((8, _RCH), jnp.int32),
        pltpu.VMEM((_RCH, D), jnp.float32),
        pltpu.VMEM((_RCH, D), jnp.float32),
        pltpu.SemaphoreType.DMA,
        pltpu.SemaphoreType.DMA,
        pltpu.SemaphoreType.DMA,
        pltpu.SemaphoreType.DMA,
    ],
)(_dispatch_body)


_JCH = 2048
_NJ = DFF // _JCH


def _mlp_body(x_ref, w1_ref, b1_ref, w2_ref, b2_ref, wv_ref, o_ref):
    e = pl.program_id(0)
    j = pl.program_id(1)

    @pl.when(e == E)
    def _():
        o_ref[...] = jnp.zeros((C, D), jnp.float32)

    @pl.when(e < E)
    def _():
        h = jnp.dot(x_ref[...].astype(jnp.bfloat16),
                    w1_ref[0].astype(jnp.bfloat16),
                    preferred_element_type=jnp.float32)
        h = jax.nn.gelu(h + b1_ref[0])
        part = jnp.dot(h.astype(jnp.bfloat16),
                       w2_ref[0].astype(jnp.bfloat16),
                       preferred_element_type=jnp.float32)

        @pl.when(j == 0)
        def _():
            o_ref[...] = part + b2_ref[0]

        @pl.when(j > 0)
        def _():
            o_ref[...] += part

        @pl.when(j == _NJ - 1)
        def _():
            o_ref[...] *= wv_ref[0]


_mlp = pl.pallas_call(
    _mlp_body,
    grid=(E + 1, _NJ),
    in_specs=[
        pl.BlockSpec((C, D), lambda e, j: (jnp.minimum(e, E - 1), 0)),
        pl.BlockSpec((1, D, _JCH), lambda e, j: (jnp.minimum(e, E - 1), 0, j)),
        pl.BlockSpec((1, 1, _JCH), lambda e, j: (jnp.minimum(e, E - 1), 0, j)),
        pl.BlockSpec((1, _JCH, D), lambda e, j: (jnp.minimum(e, E - 1), j, 0)),
        pl.BlockSpec((1, 1, D), lambda e, j: (jnp.minimum(e, E - 1), 0, 0)),
        pl.BlockSpec((1, C, 1), lambda e, j: (jnp.minimum(e, E - 1), 0, 0)),
    ],
    out_specs=pl.BlockSpec((C, D), lambda e, j: (e, 0)),
    out_shape=jax.ShapeDtypeStruct(((E + 1) * C, D), jnp.float32),
    compiler_params=pltpu.CompilerParams(
        dimension_semantics=("arbitrary", "arbitrary")),
)


_CCH = 16


def _combine_body(eout_hbm, pf_hbm, topiT_hbm, out_hbm, pfw, t0, t1, fidx,
                  bufA0, bufB0, bufA1, bufB1, gsem0, gsem1, wsem0, wsem1):
    wid = lax.axis_index("c") * 16 + lax.axis_index("s")
    base = wid * _TPW
    pltpu.sync_copy(pf_hbm.at[:, pl.ds(base, _TPW)], pfw)
    pltpu.sync_copy(topiT_hbm.at[0, pl.ds(base, _TPW)], t0)
    pltpu.sync_copy(topiT_hbm.at[1, pl.ds(base, _TPW)], t1)

    nch = _TPW // _CCH

    def fstep(i, _):
        fidx[i, :] = _sel8(pfw, t0, i)
        fidx[nch + i, :] = _sel8(pfw, t1, i)
        return 0

    lax.fori_loop(0, _TPW // L, fstep, 0)

    bufA = (bufA0, bufA1)
    bufB = (bufB0, bufB1)
    gsem = (gsem0, gsem1)
    wsem = (wsem0, wsem1)
    gd = [(pltpu.async_copy(eout_hbm.at[fidx.at[0]], bufA0, gsem0),
           pltpu.async_copy(eout_hbm.at[fidx.at[nch]], bufB0, gsem0)), None]
    wd = [None, None]
    for c in range(nch):
        b = c % 2
        nb = 1 - b
        gd[b][0].wait()
        gd[b][1].wait()
        if c + 1 < nch:
            if wd[nb] is not None:
                wd[nb].wait()
                wd[nb] = None
            gd[nb] = (
                pltpu.async_copy(eout_hbm.at[fidx.at[c + 1]], bufA[nb],
                                 gsem[nb]),
                pltpu.async_copy(eout_hbm.at[fidx.at[nch + c + 1]], bufB[nb],
                                 gsem[nb]),
            )

        def addr(r, _):
            def addv(v, __):
                bufA[b][r, pl.ds(v * L, L)] += bufB[b][r, pl.ds(v * L, L)]
                return 0
            lax.fori_loop(0, D // L, addv, 0, unroll=8)
            return 0

        lax.fori_loop(0, _CCH, addr, 0)
        wd[b] = pltpu.async_copy(bufA[b],
                                 out_hbm.at[pl.ds(base + c * _CCH, _CCH)],
                                 wsem[b])
    for w in wd:
        if w is not None:
            w.wait()


_combine = functools.partial(
    pl.kernel,
    out_type=jax.ShapeDtypeStruct((T, D), jnp.float32),
    mesh=plsc.VectorSubcoreMesh(**_MESH),
    scratch_types=[
        pltpu.VMEM((E, _TPW), jnp.int32),
        pltpu.VMEM((_TPW,), jnp.int32),
        pltpu.VMEM((_TPW,), jnp.int32),
        pltpu.VMEM((2 * (_TPW // _CCH), _CCH), jnp.int32),
        pltpu.VMEM((_CCH, D), jnp.float32),
        pltpu.VMEM((_CCH, D), jnp.float32),
        pltpu.VMEM((_CCH, D), jnp.float32),
        pltpu.VMEM((_CCH, D), jnp.float32),
        pltpu.SemaphoreType.DMA,
        pltpu.SemaphoreType.DMA,
        pltpu.SemaphoreType.DMA,
        pltpu.SemaphoreType.DMA,
    ],
)(_combine_body)


def kernel(hidden_states, wg, w1, b1, w2, b2, now_training_process):
    x = hidden_states.reshape(T, D)
    wg_p = jnp.pad(wg, ((0, 0), (0, EP - E)))
    probsT, combT, topiT, laux, cnt = _gating(x, wg_p)
    pf, wvec = _capacity(probsT, combT)
    xe = _dispatch(x, pf, topiT)
    eout = _mlp(xe, w1, b1.reshape(E, 1, DFF), w2, b2.reshape(E, 1, D),
                wvec.reshape(E, C, 1))
    out = _combine(eout, pf, topiT)
    return (out.reshape(hidden_states.shape),
            laux.reshape(()),
            cnt[0, :E])
